# Initial kernel scaffold; baseline (speedup 1.0000x reference)
import jax
import jax.numpy as jnp
from jax.experimental import pallas as pl


def kernel(x, edge_index, batch, W1, b1, W2, b2, W3, b3, Wc, bc):
    raise NotImplementedError("write your pallas kernel here")


# speedup vs baseline: 19.2673x; 19.2673x over previous
import functools

import jax
import jax.numpy as jnp
from jax import lax
from jax.experimental import pallas as pl
from jax.experimental.pallas import tpu as pltpu
from jax.experimental.pallas import tpu_sc as plsc

F32 = jnp.float32

NC = 2
NS = 16
NW = NC * NS
K = 128
NPAD = 10240


def _make_deg_kernel(NCH):
    mesh = plsc.VectorSubcoreMesh(core_axis_name="c", subcore_axis_name="s")

    @functools.partial(
        pl.kernel,
        out_type=jax.ShapeDtypeStruct((NC * NPAD,), F32),
        mesh=mesh,
        scratch_types=[
            pltpu.VMEM((NCH, K), jnp.int32),
            pltpu.VMEM((K,), F32),
            pltpu.VMEM((2048,), F32),
            pltpu.VMEM_SHARED((NPAD,), F32),
            pltpu.SemaphoreType.DMA,
        ],
    )
    def deg_kernel(dst_hbm, ones_hbm, out_hbm, dst_v, ones_v, zbuf, acc, sem):
        c = lax.axis_index("c")
        s = lax.axis_index("s")
        wid = s * NC + c
        pltpu.sync_copy(dst_hbm.at[wid], dst_v)
        pltpu.sync_copy(ones_hbm, ones_v)

        @pl.when(s == 0)
        def _():
            z16 = jnp.zeros((16,), F32)

            def zrow(r, carry):
                zbuf[pl.ds(r * 16, 16)] = z16
                return carry

            lax.fori_loop(0, 128, zrow, 0)
            for t in range(NPAD // 2048):
                pltpu.sync_copy(zbuf, acc.at[pl.ds(t * 2048, 2048)])

        plsc.subcore_barrier()

        def body(j, carry):
            pltpu.sync_copy(ones_v, acc.at[dst_v.at[j]], add=True)
            return carry

        lax.fori_loop(0, NCH, body, 0)
        plsc.subcore_barrier()

        @pl.when(s == 0)
        def _():
            pltpu.sync_copy(acc, out_hbm.at[pl.ds(c * NPAD, NPAD)])

    return deg_kernel


def _make_agg_kernel(NCH, F):
    NROW = NPAD // NS
    mesh = plsc.VectorSubcoreMesh(core_axis_name="c", subcore_axis_name="s")

    @functools.partial(
        pl.kernel,
        out_type=jax.ShapeDtypeStruct((NC * NPAD, F), F32),
        mesh=mesh,
        scratch_types=[
            pltpu.VMEM((NCH, K), jnp.int32),
            pltpu.VMEM((NCH, K), jnp.int32),
            pltpu.VMEM((K, F), F32),
            pltpu.VMEM_SHARED((NPAD, F), F32),
            pltpu.SemaphoreType.DMA,
        ],
    )
    def agg_kernel(g_hbm, src_hbm, dst_hbm, out_hbm,
                   src_v, dst_v, rows, acc, sem):
        c = lax.axis_index("c")
        s = lax.axis_index("s")
        wid = s * NC + c
        pltpu.sync_copy(src_hbm.at[wid], src_v)
        pltpu.sync_copy(dst_hbm.at[wid], dst_v)

        z16 = jnp.zeros((16,), F32)

        def zrow(r, carry):
            for cc in range(F // 16):
                rows[r, pl.ds(cc * 16, 16)] = z16
            return carry

        lax.fori_loop(0, K, zrow, 0)
        for t in range(NROW // K):
            pltpu.sync_copy(rows, acc.at[pl.ds(s * NROW + t * K, K)])
        plsc.subcore_barrier()

        def body(j, carry):
            pltpu.async_copy(g_hbm.at[src_v.at[j]], rows, sem).wait()
            pltpu.sync_copy(rows, acc.at[dst_v.at[j]], add=True)
            return carry

        lax.fori_loop(0, NCH, body, 0)
        plsc.subcore_barrier()
        pltpu.sync_copy(acc.at[pl.ds(s * NROW, NROW)],
                        out_hbm.at[pl.ds(c * NPAD + s * NROW, NROW)])

    return agg_kernel


def _mm_body(x_ref, w_ref, o_ref):
    o_ref[...] = jnp.dot(x_ref[...], w_ref[...], preferred_element_type=F32)


def _scale_body(h_ref, degp_ref, g_ref, dis_ref):
    deg = degp_ref[:, 0:1] + degp_ref[:, 1:2] + 1.0
    dis = lax.rsqrt(deg)
    dis_ref[...] = dis
    g_ref[...] = h_ref[...] * dis


def _layer_body(agg_ref, g_ref, dis_ref, b_ref, w_ref, o_ref):
    a = agg_ref[0] + agg_ref[1] + g_ref[...]
    h = jnp.maximum(a * dis_ref[...] + b_ref[...], 0.0)
    o_ref[...] = jnp.dot(h * dis_ref[...], w_ref[...], preferred_element_type=F32)


def _make_final_body(NG, BT, NB, FH):
    def final_body(agg_ref, g_ref, dis_ref, b_ref, batch_ref, wc_ref, bc_ref,
                   o_ref, acc_ref):
        i = pl.program_id(0)

        @pl.when(i == 0)
        def _():
            acc_ref[...] = jnp.zeros_like(acc_ref)

        a = agg_ref[0] + agg_ref[1] + g_ref[...]
        h = jnp.maximum(a * dis_ref[...] + b_ref[...], 0.0)[:, :FH]
        b = batch_ref[0, 0, :]
        onehot = (b[None, :] ==
                  lax.broadcasted_iota(jnp.int32, (NG, BT), 0)).astype(F32)
        hx = jnp.concatenate([h, jnp.ones((BT, 1), F32)], axis=1)
        acc_ref[...] += jnp.dot(onehot, hx, preferred_element_type=F32)

        @pl.when(i == NB - 1)
        def _():
            sums = acc_ref[:, :FH]
            cnt = jnp.maximum(acc_ref[:, FH:FH + 1], 1.0)
            pooled = sums / cnt
            logits = jnp.dot(pooled, wc_ref[...], preferred_element_type=F32) \
                + bc_ref[...]
            m = jnp.max(logits, axis=1, keepdims=True)
            lse = jnp.log(jnp.sum(jnp.exp(logits - m), axis=1, keepdims=True)) + m
            o_ref[...] = logits - lse

    return final_body


def kernel(x, edge_index, batch, W1, b1, W2, b2, W3, b3, Wc, bc):
    N, F_IN = x.shape
    E = edge_index.shape[1]
    NG = 64
    BT = 1000
    NB = N // BT
    EW = E // NW
    NCH = -(-EW // K)
    if NCH % 8:
        NCH += 8 - NCH % 8
    EWP = NCH * K
    PADW = EWP - EW

    src_w = edge_index[0].astype(jnp.int32).reshape(NW, EW)
    dst_w = edge_index[1].astype(jnp.int32).reshape(NW, EW)
    pad_src = (jnp.arange(NW * PADW, dtype=jnp.int32) % N).reshape(NW, PADW)
    pad_dst = N + (jnp.arange(NW * PADW, dtype=jnp.int32)
                   % (NPAD - N)).reshape(NW, PADW)
    src3 = jnp.concatenate([src_w, pad_src], axis=1).reshape(NW, NCH, K)
    dst3 = jnp.concatenate([dst_w, pad_dst], axis=1).reshape(NW, NCH, K)
    ones_k = jnp.ones((K,), F32)
    batch3 = batch.astype(jnp.int32).reshape(NB, 1, BT)

    W2p = jnp.pad(W2, ((0, 0), (0, 128 - W2.shape[1])))
    b2p = jnp.pad(b2, (0, 128 - b2.shape[0]))
    W3p = jnp.pad(W3, ((0, 128 - W3.shape[0]), (0, 128 - W3.shape[1])))
    b3p = jnp.pad(b3, (0, 128 - b3.shape[0]))

    degp = _make_deg_kernel(NCH)(dst3, ones_k)
    degpT = degp.reshape(NC, NPAD).T

    h1 = pl.pallas_call(
        _mm_body,
        grid=(NB,),
        in_specs=[pl.BlockSpec((BT, F_IN), lambda i: (i, 0)),
                  pl.BlockSpec((F_IN, 128), lambda i: (0, 0))],
        out_specs=pl.BlockSpec((BT, 128), lambda i: (i, 0)),
        out_shape=jax.ShapeDtypeStruct((N, 128), F32),
    )(x, W1)

    g1, dis = pl.pallas_call(
        _scale_body,
        grid=(NB,),
        in_specs=[pl.BlockSpec((BT, 128), lambda i: (i, 0)),
                  pl.BlockSpec((BT, 2), lambda i: (i, 0))],
        out_specs=[pl.BlockSpec((BT, 128), lambda i: (i, 0)),
                   pl.BlockSpec((BT, 1), lambda i: (i, 0))],
        out_shape=[jax.ShapeDtypeStruct((N, 128), F32),
                   jax.ShapeDtypeStruct((N, 1), F32)],
    )(h1, degpT)

    def layer(g, b_l, W_next):
        agg = _make_agg_kernel(NCH, 128)(g, src3, dst3)
        agg = agg.reshape(NC, NPAD, 128)
        return pl.pallas_call(
            _layer_body,
            grid=(NB,),
            in_specs=[pl.BlockSpec((2, BT, 128), lambda i: (0, i, 0)),
                      pl.BlockSpec((BT, 128), lambda i: (i, 0)),
                      pl.BlockSpec((BT, 1), lambda i: (i, 0)),
                      pl.BlockSpec((1, 128), lambda i: (0, 0)),
                      pl.BlockSpec((128, 128), lambda i: (0, 0))],
            out_specs=pl.BlockSpec((BT, 128), lambda i: (i, 0)),
            out_shape=jax.ShapeDtypeStruct((N, 128), F32),
        )(agg, g, dis, b_l.reshape(1, -1), W_next)

    g2 = layer(g1, b1, W2p)
    g3 = layer(g2, b2p, W3p)

    agg3 = _make_agg_kernel(NCH, 128)(g3, src3, dst3).reshape(NC, NPAD, 128)

    out = pl.pallas_call(
        _make_final_body(NG, BT, NB, 32),
        grid=(NB,),
        in_specs=[pl.BlockSpec((2, BT, 128), lambda i: (0, i, 0)),
                  pl.BlockSpec((BT, 128), lambda i: (i, 0)),
                  pl.BlockSpec((BT, 1), lambda i: (i, 0)),
                  pl.BlockSpec((1, 128), lambda i: (0, 0)),
                  pl.BlockSpec((1, 1, BT), lambda i: (i, 0, 0)),
                  pl.BlockSpec((32, 10), lambda i: (0, 0)),
                  pl.BlockSpec((1, 10), lambda i: (0, 0))],
        out_specs=pl.BlockSpec((NG, 10), lambda i: (0, 0)),
        out_shape=jax.ShapeDtypeStruct((NG, 10), F32),
        scratch_shapes=[pltpu.VMEM((NG, 33), F32)],
    )(agg3, g3, dis, b3p.reshape(1, -1), batch3, Wc, bc.reshape(1, -1))

    return out
